# baseline (speedup 1.0000x reference)
import jax
import jax.numpy as jnp
from jax import lax
from jax.experimental import pallas as pl
from jax.experimental.pallas import tpu as pltpu
from jax.experimental.pallas import tpu_sc as plsc

B = 8
P = 1000
PP = 1024
H = 128
D = 64

NC = 2
NS = 16
NW = NC * NS

PRED_PER_W = (B * PP) // NW
PRED_CHUNKS = PRED_PER_W // 128
HIST_PER_W = (B * H) // NW


def _sc_gather_body(w_item, ib_tbl, pred_idx, hist_idx,
                    pred_out, hist_out, ib_out,
                    pidx_v, prow_v, pbias_v, hidx_v, hrow_v, sem):
    wid = lax.axis_index("s") * NC + lax.axis_index("c")

    pltpu.sync_copy(pred_idx.at[pl.ds(wid * PRED_CHUNKS, PRED_CHUNKS)], pidx_v)
    copies = []
    for j in range(PRED_CHUNKS):
        copies.append(pltpu.async_copy(
            w_item.at[pidx_v.at[j]], prow_v.at[pl.ds(j * 128, 128)], sem))
        copies.append(pltpu.async_copy(
            ib_tbl.at[pidx_v.at[j]], pbias_v.at[j], sem))

    pltpu.sync_copy(hist_idx.at[wid], hidx_v)
    copies.append(pltpu.async_copy(w_item.at[hidx_v], hrow_v, sem))
    for c in copies:
        c.wait()

    pltpu.sync_copy(prow_v, pred_out.at[pl.ds(wid * PRED_PER_W, PRED_PER_W)])
    pltpu.sync_copy(pbias_v, ib_out.at[pl.ds(wid * PRED_CHUNKS, PRED_CHUNKS)])
    pltpu.sync_copy(hrow_v, hist_out.at[pl.ds(wid * HIST_PER_W, HIST_PER_W)])


def _make_sc_gather():
    return pl.kernel(
        _sc_gather_body,
        out_type=[
            jax.ShapeDtypeStruct((B * PP, D), jnp.float32),
            jax.ShapeDtypeStruct((B * H, D), jnp.float32),
            jax.ShapeDtypeStruct((NW * PRED_CHUNKS, 128), jnp.float32),
        ],
        mesh=plsc.VectorSubcoreMesh(core_axis_name="c", subcore_axis_name="s",
                                    num_cores=NC, num_subcores=NS),
        compiler_params=pltpu.CompilerParams(use_tc_tiling_on_sc=False),
        scratch_types=[
            pltpu.VMEM((PRED_CHUNKS, 128), jnp.int32),
            pltpu.VMEM((PRED_PER_W, D), jnp.float32),
            pltpu.VMEM((PRED_CHUNKS, 128), jnp.float32),
            pltpu.VMEM((HIST_PER_W,), jnp.int32),
            pltpu.VMEM((HIST_PER_W, D), jnp.float32),
            pltpu.SemaphoreType.DMA,
        ],
    )


def _tc_dense_body(uidx_ref, pred_ref, hist_ref, ib_ref, td_ref, wt_ref,
                   par_ref, w_user, ul_tbl, ub_tbl, out_ref,
                   urows_v, ulv, ubv, sem):
    copies = []
    for i in range(B):
        idx = uidx_ref[i]
        copies.append(pltpu.async_copy(
            w_user.at[pl.ds(idx, 1)], urows_v.at[pl.ds(i, 1)], sem))
        copies.append(pltpu.async_copy(
            ul_tbl.at[pl.ds(idx, 1)], ulv.at[pl.ds(i, 1)], sem))
        copies.append(pltpu.async_copy(
            ub_tbl.at[pl.ds(idx, 1)], ubv.at[pl.ds(i, 1)], sem))
    for c in copies:
        c.wait()

    glamb = par_ref[0]
    alpha = par_ref[1]
    beta = par_ref[2]
    gamma = par_ref[3]
    cutoff = par_ref[4]
    smooth = par_ref[5]
    force = par_ref[6]
    inv_denom = 1.0 + jnp.exp(force * smooth - smooth)
    ones_row = jnp.ones((1, D), jnp.float32)
    dn_t = (((1,), (1,)), ((), ()))
    dn_s = (((1,), (0,)), ((), ()))
    for b in range(B):
        pred = pred_ref[b]
        hist = hist_ref[b]
        u = urows_v[b:b + 1, :]
        pn = lax.dot_general(ones_row, pred * pred, dn_t,
                             preferred_element_type=jnp.float32,
                             precision=lax.Precision.HIGHEST)
        ph = lax.dot_general(hist, pred, dn_t,
                             preferred_element_type=jnp.float32,
                             precision=lax.Precision.HIGHEST)
        up = lax.dot_general(u, pred, dn_t,
                             preferred_element_type=jnp.float32,
                             precision=lax.Precision.HIGHEST)
        hn = jnp.sum(hist * hist, axis=1, keepdims=True)
        un = jnp.sum(u * u, axis=1, keepdims=True)
        dist = jnp.sqrt(jnp.maximum(hn + pn - 2.0 * ph, 0.0))
        sig = inv_denom / (1.0 + jnp.exp(force * smooth - smooth / (1.0 + dist)))
        coeff = ((glamb + ulv[b, 0])
                 * lax.rsqrt(td_ref[b:b + 1, :] + cutoff)
                 * wt_ref[b:b + 1, :])
        res = lax.dot_general(coeff, sig, dn_s,
                              preferred_element_type=jnp.float32,
                              precision=lax.Precision.HIGHEST)
        du = jnp.sqrt(jnp.maximum(un + pn - 2.0 * up, 0.0))
        outp = jnp.maximum(du - res, 0.0)
        out_ref[b:b + 1, :] = (alpha * outp + beta * outp * outp + gamma
                               + ubv[b, 0] + ib_ref[b:b + 1, :])


def _tc_dense(uidx, pred3, hist3, ib2, td, wt, params, W_user, ul_tbl, ub_tbl):
    return pl.pallas_call(
        _tc_dense_body,
        out_shape=jax.ShapeDtypeStruct((B, PP), jnp.float32),
        in_specs=[
            pl.BlockSpec(memory_space=pltpu.SMEM),
            pl.BlockSpec(memory_space=pltpu.VMEM),
            pl.BlockSpec(memory_space=pltpu.VMEM),
            pl.BlockSpec(memory_space=pltpu.VMEM),
            pl.BlockSpec(memory_space=pltpu.VMEM),
            pl.BlockSpec(memory_space=pltpu.VMEM),
            pl.BlockSpec(memory_space=pltpu.SMEM),
            pl.BlockSpec(memory_space=pltpu.MemorySpace.HBM),
            pl.BlockSpec(memory_space=pltpu.MemorySpace.HBM),
            pl.BlockSpec(memory_space=pltpu.MemorySpace.HBM),
        ],
        out_specs=pl.BlockSpec(memory_space=pltpu.VMEM),
        scratch_shapes=[
            pltpu.VMEM((B, D), jnp.float32),
            pltpu.VMEM((B, 1), jnp.float32),
            pltpu.VMEM((B, 1), jnp.float32),
            pltpu.SemaphoreType.DMA,
        ],
    )(uidx, pred3, hist3, ib2, td, wt, params, W_user, ul_tbl, ub_tbl)


def kernel(history_timedeltas, history_weights, W_user, W_item, user_lamb,
           user_bias, item_bias, global_lamb, alpha, beta, gamma, cutoff,
           smooth, force, user_index, pred_item_indices,
           history_item_indices):
    pidx = jnp.pad(pred_item_indices.astype(jnp.int32),
                   ((0, 0), (0, PP - P))).reshape(NW * PRED_CHUNKS, 128)
    hidx = history_item_indices.astype(jnp.int32).reshape(NW, HIST_PER_W)
    uidx = user_index.astype(jnp.int32)
    pred_rows, hist_rows, ib_g = _make_sc_gather()(
        W_item, item_bias.reshape(-1), pidx, hidx)
    params = jnp.stack([global_lamb, alpha, beta, gamma, cutoff, smooth,
                        force]).astype(jnp.float32)
    I_full = _tc_dense(uidx, pred_rows.reshape(B, PP, D),
                       hist_rows.reshape(B, H, D), ib_g.reshape(B, PP),
                       history_timedeltas, history_weights, params,
                       W_user, user_lamb, user_bias)
    return I_full[:, :P]


# speedup vs baseline: 1.3442x; 1.3442x over previous
import jax
import jax.numpy as jnp
from jax import lax
from jax.experimental import pallas as pl
from jax.experimental.pallas import tpu as pltpu
from jax.experimental.pallas import tpu_sc as plsc

B = 8
P = 1000
PP = 1024
H = 128
D = 64

NC = 2
NS = 16
NW = NC * NS

NBLK = B * PP // 128
BLK_PER_W = NBLK // NW
PRED_PER_W = BLK_PER_W * 128
HIST_PER_W = (B * H) // NW


def _sc_gather_body(w_item, ib_tbl, ul_tbl, ub_tbl,
                    pred_idx, ib_idx, hist_idx, user_idx,
                    pred_out, hist_out, ib_out, ul_out, ub_out,
                    pidx_v, prow_v, bidx_v, pbias_v, hidx_v, hrow_v,
                    uidx_v, ulv, ubv, sem, semu):
    wid = lax.axis_index("s") * NC + lax.axis_index("c")

    pltpu.sync_copy(pred_idx.at[pl.ds(wid * BLK_PER_W, BLK_PER_W)], pidx_v)
    pltpu.sync_copy(ib_idx.at[pl.ds(wid * BLK_PER_W, BLK_PER_W)], bidx_v)
    bh = lax.div(wid, 4)
    cb = lax.rem(wid, 4) * HIST_PER_W
    pltpu.sync_copy(hist_idx.at[pl.ds(bh, 1), pl.ds(cb, HIST_PER_W)], hidx_v)

    copies = []
    for j in range(BLK_PER_W):
        copies.append(pltpu.async_copy(
            w_item.at[pidx_v.at[j]], prow_v.at[pl.ds(j * 128, 128)], sem))
        copies.append(pltpu.async_copy(
            ib_tbl.at[bidx_v.at[j]], pbias_v.at[j], sem))
    copies.append(pltpu.async_copy(w_item.at[hidx_v.at[0]], hrow_v, sem))

    @pl.when(wid == 0)
    def _():
        pltpu.sync_copy(user_idx, uidx_v)
        cl = pltpu.async_copy(ul_tbl.at[uidx_v], ulv, semu)
        cb2 = pltpu.async_copy(ub_tbl.at[uidx_v], ubv, semu)
        cl.wait()
        cb2.wait()
        pltpu.sync_copy(ulv, ul_out)
        pltpu.sync_copy(ubv, ub_out)

    for c in copies:
        c.wait()
    pltpu.sync_copy(prow_v, pred_out.at[pl.ds(wid * PRED_PER_W, PRED_PER_W)])
    pltpu.sync_copy(pbias_v, ib_out.at[pl.ds(wid * BLK_PER_W, BLK_PER_W)])
    pltpu.sync_copy(hrow_v, hist_out.at[pl.ds(wid * HIST_PER_W, HIST_PER_W)])


def _make_sc_gather():
    return pl.kernel(
        _sc_gather_body,
        out_type=[
            jax.ShapeDtypeStruct((B * PP, D), jnp.float32),
            jax.ShapeDtypeStruct((B * H, D), jnp.float32),
            jax.ShapeDtypeStruct((NBLK, 128), jnp.float32),
            jax.ShapeDtypeStruct((B,), jnp.float32),
            jax.ShapeDtypeStruct((B,), jnp.float32),
        ],
        mesh=plsc.VectorSubcoreMesh(core_axis_name="c", subcore_axis_name="s",
                                    num_cores=NC, num_subcores=NS),
        compiler_params=pltpu.CompilerParams(use_tc_tiling_on_sc=False),
        scratch_types=[
            pltpu.VMEM((BLK_PER_W, 128), jnp.int32),
            pltpu.VMEM((PRED_PER_W, D), jnp.float32),
            pltpu.VMEM((BLK_PER_W, 128), jnp.int32),
            pltpu.VMEM((BLK_PER_W, 128), jnp.float32),
            pltpu.VMEM((1, HIST_PER_W), jnp.int32),
            pltpu.VMEM((HIST_PER_W, D), jnp.float32),
            pltpu.VMEM((B,), jnp.int32),
            pltpu.VMEM((B,), jnp.float32),
            pltpu.VMEM((B,), jnp.float32),
            pltpu.SemaphoreType.DMA,
            pltpu.SemaphoreType.DMA,
        ],
    )


def _scs_user_body(w_user, user_idx, user_out, uidx_s, urow_s):
    cid = lax.axis_index("c")

    @pl.when(cid == 0)
    def _():
        pltpu.sync_copy(user_idx, uidx_s)
        for i in range(B):
            sidx = uidx_s[i]
            pltpu.sync_copy(w_user.at[pl.ds(sidx, 1)], urow_s)
            pltpu.sync_copy(urow_s, user_out.at[pl.ds(i, 1)])


def _make_scs_user():
    return pl.kernel(
        _scs_user_body,
        out_type=[jax.ShapeDtypeStruct((B, D), jnp.float32)],
        mesh=plsc.ScalarSubcoreMesh(axis_name="c", num_cores=NC),
        scratch_types=[
            pltpu.SMEM((B,), jnp.int32),
            pltpu.SMEM((1, D), jnp.float32),
        ],
    )


def _tc_dense_body(p2_ref, hist_ref, user_ref, ib_ref, td_ref, wt_ref,
                   ul_ref, ub_ref, par_ref, out_ref):
    glamb = par_ref[0]
    alpha = par_ref[1]
    beta = par_ref[2]
    gamma = par_ref[3]
    cutoff = par_ref[4]
    smooth = par_ref[5]
    force = par_ref[6]
    inv_denom = 1.0 + jnp.exp(force * smooth - smooth)
    zrow = jnp.zeros((1, D), jnp.float32)
    orow = jnp.ones((1, D), jnp.float32)
    zh = jnp.zeros((H, D), jnp.float32)
    dn_t = (((1,), (1,)), ((), ()))
    PH = PP // 2
    for b in range(B):
        p2 = p2_ref[b]
        hist = hist_ref[b]
        u = user_ref[b:b + 1, :]
        hlr = jnp.concatenate(
            [jnp.concatenate([hist, zh], axis=1),
             jnp.concatenate([zh, hist], axis=1)], axis=0)
        G = lax.dot_general(hlr, p2, dn_t,
                            preferred_element_type=jnp.float32,
                            precision=lax.Precision.HIGHEST)
        aux = jnp.concatenate(
            [jnp.concatenate([orow, zrow], axis=1),
             jnp.concatenate([zrow, orow], axis=1)], axis=0)
        S = lax.dot_general(aux, p2 * p2, dn_t,
                            preferred_element_type=jnp.float32,
                            precision=lax.Precision.HIGHEST)
        uax = jnp.concatenate(
            [jnp.concatenate([u, zrow], axis=1),
             jnp.concatenate([zrow, u], axis=1)], axis=0)
        U = lax.dot_general(uax, p2, dn_t,
                            preferred_element_type=jnp.float32,
                            precision=lax.Precision.HIGHEST)
        hn = jnp.sum(hist * hist, axis=1, keepdims=True)
        un = jnp.sum(u * u, axis=1, keepdims=True)
        hn2 = jnp.concatenate([hn, hn], axis=0)
        pn2 = jnp.concatenate(
            [jnp.broadcast_to(S[0:1], (H, PH)),
             jnp.broadcast_to(S[1:2], (H, PH))], axis=0)
        dist = jnp.sqrt(jnp.maximum(hn2 + pn2 - 2.0 * G, 0.0))
        sig = inv_denom / (1.0 + jnp.exp(force * smooth - smooth / (1.0 + dist)))
        coeff = ((glamb + ul_ref[b])
                 * lax.rsqrt(td_ref[b:b + 1, :] + cutoff)
                 * wt_ref[b:b + 1, :])
        dn_s = (((1,), (0,)), ((), ()))
        resE = lax.dot_general(coeff, sig[0:H], dn_s,
                               preferred_element_type=jnp.float32,
                               precision=lax.Precision.HIGHEST)
        resO = lax.dot_general(coeff, sig[H:2 * H], dn_s,
                               preferred_element_type=jnp.float32,
                               precision=lax.Precision.HIGHEST)
        duE = jnp.sqrt(jnp.maximum(un + S[0:1] - 2.0 * U[0:1], 0.0))
        duO = jnp.sqrt(jnp.maximum(un + S[1:2] - 2.0 * U[1:2], 0.0))
        outE = jnp.maximum(duE - resE, 0.0)
        outO = jnp.maximum(duO - resO, 0.0)
        ibE = jnp.concatenate(
            [ib_ref[q * 16 + b:q * 16 + b + 1, :] for q in range(4)], axis=1)
        ibO = jnp.concatenate(
            [ib_ref[q * 16 + 8 + b:q * 16 + 8 + b + 1, :] for q in range(4)],
            axis=1)
        com = gamma + ub_ref[b]
        out_ref[b:b + 1, 0:PH] = (alpha * outE + beta * outE * outE
                                  + com + ibE)
        out_ref[b:b + 1, PH:PP] = (alpha * outO + beta * outO * outO
                                   + com + ibO)


def _tc_dense(p2, hist3, user_rows, ib_g, td, wt, ul_g, ub_g, params):
    return pl.pallas_call(
        _tc_dense_body,
        out_shape=jax.ShapeDtypeStruct((B, PP), jnp.float32),
        in_specs=[
            pl.BlockSpec(memory_space=pltpu.VMEM),
            pl.BlockSpec(memory_space=pltpu.VMEM),
            pl.BlockSpec(memory_space=pltpu.VMEM),
            pl.BlockSpec(memory_space=pltpu.VMEM),
            pl.BlockSpec(memory_space=pltpu.VMEM),
            pl.BlockSpec(memory_space=pltpu.VMEM),
            pl.BlockSpec(memory_space=pltpu.SMEM),
            pl.BlockSpec(memory_space=pltpu.SMEM),
            pl.BlockSpec(memory_space=pltpu.SMEM),
        ],
        out_specs=pl.BlockSpec(memory_space=pltpu.VMEM),
    )(p2, hist3, user_rows, ib_g, td, wt, ul_g, ub_g, params)


def kernel(history_timedeltas, history_weights, W_user, W_item, user_lamb,
           user_bias, item_bias, global_lamb, alpha, beta, gamma, cutoff,
           smooth, force, user_index, pred_item_indices,
           history_item_indices):
    pidx = jnp.pad(pred_item_indices.astype(jnp.int32), ((0, 0), (0, PP - P)))
    pidx_bc = pidx.reshape(NBLK, 128)
    ib_idx = (pidx.reshape(B, 4, 128, 2).transpose(1, 3, 0, 2)
              .reshape(NBLK, 128))
    hidx = history_item_indices.astype(jnp.int32)
    uidx = user_index.astype(jnp.int32)
    pred_rows, hist_rows, ib_g, ul_g, ub_g = _make_sc_gather()(
        W_item, item_bias.reshape(-1), user_lamb.reshape(-1),
        user_bias.reshape(-1), pidx_bc, ib_idx, hidx, uidx)
    user_rows, = _make_scs_user()(W_user, uidx)
    params = jnp.stack([global_lamb, alpha, beta, gamma, cutoff, smooth,
                        force]).astype(jnp.float32)
    out = _tc_dense(pred_rows.reshape(B, PP // 2, 128),
                    hist_rows.reshape(B, H, D), user_rows, ib_g,
                    history_timedeltas, history_weights, ul_g, ub_g, params)
    I_full = out.reshape(B, 2, PP // 2).transpose(0, 2, 1).reshape(B, PP)
    return I_full[:, :P]


# speedup vs baseline: 1.3468x; 1.0019x over previous
import jax
import jax.numpy as jnp
from jax import lax
from jax.experimental import pallas as pl
from jax.experimental.pallas import tpu as pltpu
from jax.experimental.pallas import tpu_sc as plsc

B = 8
P = 1000
PP = 1024
H = 128
D = 64

NC = 2
NS = 16
NW = NC * NS

NBLK = B * PP // 128
BLK_PER_W = NBLK // NW
PRED_PER_W = BLK_PER_W * 128
HIST_PER_W = (B * H) // NW


def _sc_gather_body(w_item, ul_tbl, ub_tbl,
                    pred_idx, hist_idx, user_idx,
                    pred_out, hist_out, ul_out, ub_out,
                    pidx_v, prow_v, hidx_v, hrow_v,
                    uidx_v, ulv, ubv, sem, semu):
    wid = lax.axis_index("s") * NC + lax.axis_index("c")

    pltpu.sync_copy(pred_idx.at[pl.ds(wid * BLK_PER_W, BLK_PER_W)], pidx_v)
    bh = lax.div(wid, 4)
    cb = lax.rem(wid, 4) * HIST_PER_W
    pltpu.sync_copy(hist_idx.at[pl.ds(bh, 1), pl.ds(cb, HIST_PER_W)], hidx_v)

    copies = []
    for j in range(BLK_PER_W):
        copies.append(pltpu.async_copy(
            w_item.at[pidx_v.at[j]], prow_v.at[pl.ds(j * 128, 128)], sem))
    copies.append(pltpu.async_copy(w_item.at[hidx_v.at[0]], hrow_v, sem))

    @pl.when(wid == 0)
    def _():
        pltpu.sync_copy(user_idx, uidx_v)
        cl = pltpu.async_copy(ul_tbl.at[uidx_v], ulv, semu)
        cb2 = pltpu.async_copy(ub_tbl.at[uidx_v], ubv, semu)
        cl.wait()
        cb2.wait()
        pltpu.sync_copy(ulv, ul_out)
        pltpu.sync_copy(ubv, ub_out)

    for c in copies:
        c.wait()
    pltpu.sync_copy(prow_v, pred_out.at[pl.ds(wid * PRED_PER_W, PRED_PER_W)])
    pltpu.sync_copy(hrow_v, hist_out.at[pl.ds(wid * HIST_PER_W, HIST_PER_W)])


def _make_sc_gather():
    return pl.kernel(
        _sc_gather_body,
        out_type=[
            jax.ShapeDtypeStruct((B * PP, D), jnp.float32),
            jax.ShapeDtypeStruct((B * H, D), jnp.float32),
            jax.ShapeDtypeStruct((B,), jnp.float32),
            jax.ShapeDtypeStruct((B,), jnp.float32),
        ],
        mesh=plsc.VectorSubcoreMesh(core_axis_name="c", subcore_axis_name="s",
                                    num_cores=NC, num_subcores=NS),
        compiler_params=pltpu.CompilerParams(use_tc_tiling_on_sc=False),
        scratch_types=[
            pltpu.VMEM((BLK_PER_W, 128), jnp.int32),
            pltpu.VMEM((PRED_PER_W, D), jnp.float32),
            pltpu.VMEM((1, HIST_PER_W), jnp.int32),
            pltpu.VMEM((HIST_PER_W, D), jnp.float32),
            pltpu.VMEM((B,), jnp.int32),
            pltpu.VMEM((B,), jnp.float32),
            pltpu.VMEM((B,), jnp.float32),
            pltpu.SemaphoreType.DMA,
            pltpu.SemaphoreType.DMA,
        ],
    )


def _sc_bias_body(ib_tbl, ib_idx, ib_out, bidx_v, pbias_v, sem):
    wid = lax.axis_index("s") * NC + lax.axis_index("c")
    pltpu.sync_copy(ib_idx.at[pl.ds(wid * BLK_PER_W, BLK_PER_W)], bidx_v)
    copies = []
    for j in range(BLK_PER_W):
        copies.append(pltpu.async_copy(ib_tbl.at[bidx_v.at[j]],
                                       pbias_v.at[j], sem))
    for c in copies:
        c.wait()
    pltpu.sync_copy(pbias_v, ib_out.at[pl.ds(wid * BLK_PER_W, BLK_PER_W)])


def _make_sc_bias():
    return pl.kernel(
        _sc_bias_body,
        out_type=[jax.ShapeDtypeStruct((NBLK, 128), jnp.float32)],
        mesh=plsc.VectorSubcoreMesh(core_axis_name="c", subcore_axis_name="s",
                                    num_cores=NC, num_subcores=NS),
        compiler_params=pltpu.CompilerParams(use_tc_tiling_on_sc=False),
        scratch_types=[
            pltpu.VMEM((BLK_PER_W, 128), jnp.int32),
            pltpu.VMEM((BLK_PER_W, 128), jnp.float32),
            pltpu.SemaphoreType.DMA,
        ],
    )


def _scs_user_body(w_user, user_idx, user_out, uidx_s, urows_s, sem):
    cid = lax.axis_index("c")

    @pl.when(cid == 0)
    def _():
        pltpu.sync_copy(user_idx, uidx_s)
        copies = []
        for i in range(B):
            sidx = uidx_s[i]
            copies.append(pltpu.async_copy(
                w_user.at[pl.ds(sidx, 1)], urows_s.at[pl.ds(i, 1)], sem))
        for c in copies:
            c.wait()
        pltpu.sync_copy(urows_s, user_out)


def _make_scs_user():
    return pl.kernel(
        _scs_user_body,
        out_type=[jax.ShapeDtypeStruct((B, D), jnp.float32)],
        mesh=plsc.ScalarSubcoreMesh(axis_name="c", num_cores=NC),
        compiler_params=pltpu.CompilerParams(use_tc_tiling_on_sc=True),
        scratch_types=[
            pltpu.SMEM((B,), jnp.int32),
            pltpu.SMEM((B, D), jnp.float32),
            pltpu.SemaphoreType.DMA,
        ],
    )


def _tc_dense_body(p2_ref, hist_ref, user_ref, ib_ref, td_ref, wt_ref,
                   ul_ref, ub_ref, par_ref, out_ref):
    glamb = par_ref[0]
    alpha = par_ref[1]
    beta = par_ref[2]
    gamma = par_ref[3]
    cutoff = par_ref[4]
    smooth = par_ref[5]
    force = par_ref[6]
    inv_denom = 1.0 + jnp.exp(force * smooth - smooth)
    zrow = jnp.zeros((1, D), jnp.float32)
    orow = jnp.ones((1, D), jnp.float32)
    zh = jnp.zeros((H, D), jnp.float32)
    dn_t = (((1,), (1,)), ((), ()))
    PH = PP // 2
    for b in range(B):
        p2 = p2_ref[b]
        hist = hist_ref[b]
        u = user_ref[b:b + 1, :]
        hlr = jnp.concatenate(
            [jnp.concatenate([hist, zh], axis=1),
             jnp.concatenate([zh, hist], axis=1)], axis=0)
        G = lax.dot_general(hlr, p2, dn_t,
                            preferred_element_type=jnp.float32,
                            precision=lax.Precision.HIGHEST)
        aux = jnp.concatenate(
            [jnp.concatenate([orow, zrow], axis=1),
             jnp.concatenate([zrow, orow], axis=1)], axis=0)
        S = lax.dot_general(aux, p2 * p2, dn_t,
                            preferred_element_type=jnp.float32,
                            precision=lax.Precision.HIGHEST)
        uax = jnp.concatenate(
            [jnp.concatenate([u, zrow], axis=1),
             jnp.concatenate([zrow, u], axis=1)], axis=0)
        U = lax.dot_general(uax, p2, dn_t,
                            preferred_element_type=jnp.float32,
                            precision=lax.Precision.HIGHEST)
        hn = jnp.sum(hist * hist, axis=1, keepdims=True)
        un = jnp.sum(u * u, axis=1, keepdims=True)
        hn2 = jnp.concatenate([hn, hn], axis=0)
        pn2 = jnp.concatenate(
            [jnp.broadcast_to(S[0:1], (H, PH)),
             jnp.broadcast_to(S[1:2], (H, PH))], axis=0)
        dist = jnp.sqrt(jnp.maximum(hn2 + pn2 - 2.0 * G, 0.0))
        sig = inv_denom / (1.0 + jnp.exp(force * smooth - smooth / (1.0 + dist)))
        coeff = ((glamb + ul_ref[b])
                 * lax.rsqrt(td_ref[b:b + 1, :] + cutoff)
                 * wt_ref[b:b + 1, :])
        dn_s = (((1,), (0,)), ((), ()))
        resE = lax.dot_general(coeff, sig[0:H], dn_s,
                               preferred_element_type=jnp.float32,
                               precision=lax.Precision.HIGHEST)
        resO = lax.dot_general(coeff, sig[H:2 * H], dn_s,
                               preferred_element_type=jnp.float32,
                               precision=lax.Precision.HIGHEST)
        duE = jnp.sqrt(jnp.maximum(un + S[0:1] - 2.0 * U[0:1], 0.0))
        duO = jnp.sqrt(jnp.maximum(un + S[1:2] - 2.0 * U[1:2], 0.0))
        outE = jnp.maximum(duE - resE, 0.0)
        outO = jnp.maximum(duO - resO, 0.0)
        ibE = jnp.concatenate(
            [ib_ref[q * 16 + b:q * 16 + b + 1, :] for q in range(4)], axis=1)
        ibO = jnp.concatenate(
            [ib_ref[q * 16 + 8 + b:q * 16 + 8 + b + 1, :] for q in range(4)],
            axis=1)
        com = gamma + ub_ref[b]
        out_ref[b:b + 1, 0:PH] = (alpha * outE + beta * outE * outE
                                  + com + ibE)
        out_ref[b:b + 1, PH:PP] = (alpha * outO + beta * outO * outO
                                   + com + ibO)


def _tc_dense(p2, hist3, user_rows, ib_g, td, wt, ul_g, ub_g, params):
    return pl.pallas_call(
        _tc_dense_body,
        out_shape=jax.ShapeDtypeStruct((B, PP), jnp.float32),
        in_specs=[
            pl.BlockSpec(memory_space=pltpu.VMEM),
            pl.BlockSpec(memory_space=pltpu.VMEM),
            pl.BlockSpec(memory_space=pltpu.VMEM),
            pl.BlockSpec(memory_space=pltpu.VMEM),
            pl.BlockSpec(memory_space=pltpu.VMEM),
            pl.BlockSpec(memory_space=pltpu.VMEM),
            pl.BlockSpec(memory_space=pltpu.SMEM),
            pl.BlockSpec(memory_space=pltpu.SMEM),
            pl.BlockSpec(memory_space=pltpu.SMEM),
        ],
        out_specs=pl.BlockSpec(memory_space=pltpu.VMEM),
    )(p2, hist3, user_rows, ib_g, td, wt, ul_g, ub_g, params)


def kernel(history_timedeltas, history_weights, W_user, W_item, user_lamb,
           user_bias, item_bias, global_lamb, alpha, beta, gamma, cutoff,
           smooth, force, user_index, pred_item_indices,
           history_item_indices):
    pidx = jnp.pad(pred_item_indices.astype(jnp.int32), ((0, 0), (0, PP - P)))
    pidx_bc = pidx.reshape(NBLK, 128)
    ib_idx = (pidx.reshape(B, 4, 128, 2).transpose(1, 3, 0, 2)
              .reshape(NBLK, 128))
    hidx = history_item_indices.astype(jnp.int32)
    uidx = user_index.astype(jnp.int32)
    pred_rows, hist_rows, ul_g, ub_g = _make_sc_gather()(
        W_item, user_lamb.reshape(-1), user_bias.reshape(-1),
        pidx_bc, hidx, uidx)
    ib_g, = _make_sc_bias()(item_bias.reshape(-1), ib_idx)
    user_rows, = _make_scs_user()(W_user, uidx)
    params = jnp.stack([global_lamb, alpha, beta, gamma, cutoff, smooth,
                        force]).astype(jnp.float32)
    out = _tc_dense(pred_rows.reshape(B, PP // 2, 128),
                    hist_rows.reshape(B, H, D), user_rows, ib_g,
                    history_timedeltas, history_weights, ul_g, ub_g, params)
    I_full = out.reshape(B, 2, PP // 2).transpose(0, 2, 1).reshape(B, PP)
    return I_full[:, :P]


# speedup vs baseline: 1.7414x; 1.2930x over previous
import jax
import jax.numpy as jnp
from jax import lax
from jax.experimental import pallas as pl
from jax.experimental.pallas import tpu as pltpu
from jax.experimental.pallas import tpu_sc as plsc

B = 8
P = 1000
PP = 1024
H = 128
D = 64

NC = 2
NS = 16
NW = NC * NS

NBLK = B * PP // 128
BLK_PER_W = NBLK // NW
PRED_PER_W = BLK_PER_W * 128
HIST_PER_W = (B * H) // NW


def _sc_gather_body(w_item, ul_tbl, ub_tbl,
                    pred_idx, hist_idx, user_idx,
                    pred_out, hist_out, ul_out, ub_out,
                    pidx_v, prow_v, hidx_v, hrow_v,
                    uidx_v, ulv, ubv, sem, semu):
    wid = lax.axis_index("s") * NC + lax.axis_index("c")

    pltpu.sync_copy(pred_idx.at[pl.ds(wid * BLK_PER_W, BLK_PER_W)], pidx_v)
    bh = lax.div(wid, 4)
    cb = lax.rem(wid, 4) * HIST_PER_W
    pltpu.sync_copy(hist_idx.at[pl.ds(bh, 1), pl.ds(cb, HIST_PER_W)], hidx_v)

    copies = []
    for j in range(BLK_PER_W):
        copies.append(pltpu.async_copy(
            w_item.at[pidx_v.at[j]], prow_v.at[pl.ds(j * 128, 128)], sem))
    copies.append(pltpu.async_copy(w_item.at[hidx_v.at[0]], hrow_v, sem))

    @pl.when(wid == 0)
    def _():
        pltpu.sync_copy(user_idx, uidx_v)
        cl = pltpu.async_copy(ul_tbl.at[uidx_v], ulv, semu)
        cb2 = pltpu.async_copy(ub_tbl.at[uidx_v], ubv, semu)
        cl.wait()
        cb2.wait()
        pltpu.sync_copy(ulv, ul_out)
        pltpu.sync_copy(ubv, ub_out)

    for c in copies:
        c.wait()
    pltpu.sync_copy(prow_v, pred_out.at[pl.ds(wid * PRED_PER_W, PRED_PER_W)])
    pltpu.sync_copy(hrow_v, hist_out.at[pl.ds(wid * HIST_PER_W, HIST_PER_W)])


def _make_sc_gather():
    return pl.kernel(
        _sc_gather_body,
        out_type=[
            jax.ShapeDtypeStruct((B * PP, D), jnp.float32),
            jax.ShapeDtypeStruct((B * H, D), jnp.float32),
            jax.ShapeDtypeStruct((B,), jnp.float32),
            jax.ShapeDtypeStruct((B,), jnp.float32),
        ],
        mesh=plsc.VectorSubcoreMesh(core_axis_name="c", subcore_axis_name="s",
                                    num_cores=NC, num_subcores=NS),
        compiler_params=pltpu.CompilerParams(use_tc_tiling_on_sc=False),
        scratch_types=[
            pltpu.VMEM((BLK_PER_W, 128), jnp.int32),
            pltpu.VMEM((PRED_PER_W, D), jnp.float32),
            pltpu.VMEM((1, HIST_PER_W), jnp.int32),
            pltpu.VMEM((HIST_PER_W, D), jnp.float32),
            pltpu.VMEM((B,), jnp.int32),
            pltpu.VMEM((B,), jnp.float32),
            pltpu.VMEM((B,), jnp.float32),
            pltpu.SemaphoreType.DMA,
            pltpu.SemaphoreType.DMA,
        ],
    )


def _sc_bias_body(ib_tbl, ib_idx, ib_out, bidx_v, pbias_v, sem):
    wid = lax.axis_index("s") * NC + lax.axis_index("c")
    pltpu.sync_copy(ib_idx.at[pl.ds(wid * BLK_PER_W, BLK_PER_W)], bidx_v)
    copies = []
    for j in range(BLK_PER_W):
        copies.append(pltpu.async_copy(ib_tbl.at[bidx_v.at[j]],
                                       pbias_v.at[j], sem))
    for c in copies:
        c.wait()
    pltpu.sync_copy(pbias_v, ib_out.at[pl.ds(wid * BLK_PER_W, BLK_PER_W)])


def _make_sc_bias():
    return pl.kernel(
        _sc_bias_body,
        out_type=[jax.ShapeDtypeStruct((NBLK, 128), jnp.float32)],
        mesh=plsc.VectorSubcoreMesh(core_axis_name="c", subcore_axis_name="s",
                                    num_cores=NC, num_subcores=NS),
        compiler_params=pltpu.CompilerParams(use_tc_tiling_on_sc=False),
        scratch_types=[
            pltpu.VMEM((BLK_PER_W, 128), jnp.int32),
            pltpu.VMEM((BLK_PER_W, 128), jnp.float32),
            pltpu.SemaphoreType.DMA,
        ],
    )


def _tc_dense_body(p2_ref, hist_ref, w_userT, uidx_ref, ib_ref, td_ref,
                   wt_ref, ul_ref, ub_ref, par_ref, out_ref,
                   ucol0, ucol1, ucol2, ucol3, ucol4, ucol5, ucol6, ucol7,
                   sem):
    ucols = [ucol0, ucol1, ucol2, ucol3, ucol4, ucol5, ucol6, ucol7]
    copies = []
    for i in range(B):
        base = pl.multiple_of((uidx_ref[i] // 128) * 128, 128)
        copies.append(pltpu.async_copy(
            w_userT.at[:, pl.ds(base, 128)], ucols[i], sem))
    for c in copies:
        c.wait()
    lanes = jax.lax.broadcasted_iota(jnp.int32, (1, 128), 1)
    dn_t0 = (((1,), (1,)), ((), ()))
    user_all = jnp.concatenate(
        [lax.dot_general(
            (lanes == (uidx_ref[i] % 128)).astype(jnp.float32),
            ucols[i][:, :], dn_t0,
            preferred_element_type=jnp.float32,
            precision=lax.Precision.HIGHEST)
         for i in range(B)], axis=0)
    _tc_math(p2_ref, hist_ref, user_all, ib_ref, td_ref, wt_ref,
             ul_ref, ub_ref, par_ref, out_ref)


def _tc_math(p2_ref, hist_ref, user_all, ib_ref, td_ref, wt_ref,
             ul_ref, ub_ref, par_ref, out_ref):
    glamb = par_ref[0]
    alpha = par_ref[1]
    beta = par_ref[2]
    gamma = par_ref[3]
    cutoff = par_ref[4]
    smooth = par_ref[5]
    force = par_ref[6]
    inv_denom = 1.0 + jnp.exp(force * smooth - smooth)
    zrow = jnp.zeros((1, D), jnp.float32)
    orow = jnp.ones((1, D), jnp.float32)
    zh = jnp.zeros((H, D), jnp.float32)
    dn_t = (((1,), (1,)), ((), ()))
    PH = PP // 2
    for b in range(B):
        p2 = p2_ref[b]
        hist = hist_ref[b]
        u = user_all[b:b + 1, :]
        hlr = jnp.concatenate(
            [jnp.concatenate([hist, zh], axis=1),
             jnp.concatenate([zh, hist], axis=1)], axis=0)
        G = lax.dot_general(hlr, p2, dn_t,
                            preferred_element_type=jnp.float32,
                            precision=lax.Precision.HIGHEST)
        aux = jnp.concatenate(
            [jnp.concatenate([orow, zrow], axis=1),
             jnp.concatenate([zrow, orow], axis=1)], axis=0)
        S = lax.dot_general(aux, p2 * p2, dn_t,
                            preferred_element_type=jnp.float32,
                            precision=lax.Precision.HIGHEST)
        uax = jnp.concatenate(
            [jnp.concatenate([u, zrow], axis=1),
             jnp.concatenate([zrow, u], axis=1)], axis=0)
        U = lax.dot_general(uax, p2, dn_t,
                            preferred_element_type=jnp.float32,
                            precision=lax.Precision.HIGHEST)
        hn = jnp.sum(hist * hist, axis=1, keepdims=True)
        un = jnp.sum(u * u, axis=1, keepdims=True)
        hn2 = jnp.concatenate([hn, hn], axis=0)
        pn2 = jnp.concatenate(
            [jnp.broadcast_to(S[0:1], (H, PH)),
             jnp.broadcast_to(S[1:2], (H, PH))], axis=0)
        dist = jnp.sqrt(jnp.maximum(hn2 + pn2 - 2.0 * G, 0.0))
        sig = inv_denom / (1.0 + jnp.exp(force * smooth - smooth / (1.0 + dist)))
        coeff = ((glamb + ul_ref[b])
                 * lax.rsqrt(td_ref[b:b + 1, :] + cutoff)
                 * wt_ref[b:b + 1, :])
        dn_s = (((1,), (0,)), ((), ()))
        resE = lax.dot_general(coeff, sig[0:H], dn_s,
                               preferred_element_type=jnp.float32,
                               precision=lax.Precision.HIGHEST)
        resO = lax.dot_general(coeff, sig[H:2 * H], dn_s,
                               preferred_element_type=jnp.float32,
                               precision=lax.Precision.HIGHEST)
        duE = jnp.sqrt(jnp.maximum(un + S[0:1] - 2.0 * U[0:1], 0.0))
        duO = jnp.sqrt(jnp.maximum(un + S[1:2] - 2.0 * U[1:2], 0.0))
        outE = jnp.maximum(duE - resE, 0.0)
        outO = jnp.maximum(duO - resO, 0.0)
        ibE = jnp.concatenate(
            [ib_ref[q * 16 + b:q * 16 + b + 1, :] for q in range(4)], axis=1)
        ibO = jnp.concatenate(
            [ib_ref[q * 16 + 8 + b:q * 16 + 8 + b + 1, :] for q in range(4)],
            axis=1)
        com = gamma + ub_ref[b]
        out_ref[b:b + 1, 0:PH] = (alpha * outE + beta * outE * outE
                                  + com + ibE)
        out_ref[b:b + 1, PH:PP] = (alpha * outO + beta * outO * outO
                                   + com + ibO)


def _tc_dense(p2, hist3, w_userT, uidx, ib_g, td, wt, ul_g, ub_g, params):
    return pl.pallas_call(
        _tc_dense_body,
        out_shape=jax.ShapeDtypeStruct((B, PP), jnp.float32),
        in_specs=[
            pl.BlockSpec(memory_space=pltpu.VMEM),
            pl.BlockSpec(memory_space=pltpu.VMEM),
            pl.BlockSpec(memory_space=pltpu.MemorySpace.HBM),
            pl.BlockSpec(memory_space=pltpu.SMEM),
            pl.BlockSpec(memory_space=pltpu.VMEM),
            pl.BlockSpec(memory_space=pltpu.VMEM),
            pl.BlockSpec(memory_space=pltpu.VMEM),
            pl.BlockSpec(memory_space=pltpu.SMEM),
            pl.BlockSpec(memory_space=pltpu.SMEM),
            pl.BlockSpec(memory_space=pltpu.SMEM),
        ],
        out_specs=pl.BlockSpec(memory_space=pltpu.VMEM),
        scratch_shapes=[pltpu.VMEM((D, 128), jnp.float32) for _ in range(B)]
        + [pltpu.SemaphoreType.DMA],
    )(p2, hist3, w_userT, uidx, ib_g, td, wt, ul_g, ub_g, params)


def kernel(history_timedeltas, history_weights, W_user, W_item, user_lamb,
           user_bias, item_bias, global_lamb, alpha, beta, gamma, cutoff,
           smooth, force, user_index, pred_item_indices,
           history_item_indices):
    pidx = jnp.pad(pred_item_indices.astype(jnp.int32), ((0, 0), (0, PP - P)))
    pidx_bc = pidx.reshape(NBLK, 128)
    ib_idx = (pidx.reshape(B, 4, 128, 2).transpose(1, 3, 0, 2)
              .reshape(NBLK, 128))
    hidx = history_item_indices.astype(jnp.int32)
    uidx = user_index.astype(jnp.int32)
    pred_rows, hist_rows, ul_g, ub_g = _make_sc_gather()(
        W_item, user_lamb.reshape(-1), user_bias.reshape(-1),
        pidx_bc, hidx, uidx)
    ib_g, = _make_sc_bias()(item_bias.reshape(-1), ib_idx)
    params = jnp.stack([global_lamb, alpha, beta, gamma, cutoff, smooth,
                        force]).astype(jnp.float32)
    out = _tc_dense(pred_rows.reshape(B, PP // 2, 128),
                    hist_rows.reshape(B, H, D), W_user.T, uidx, ib_g,
                    history_timedeltas, history_weights, ul_g, ub_g, params)
    I_full = out.reshape(B, 2, PP // 2).transpose(0, 2, 1).reshape(B, PP)
    return I_full[:, :P]


# speedup vs baseline: 2.0544x; 1.1798x over previous
import jax
import jax.numpy as jnp
from jax import lax
from jax.experimental import pallas as pl
from jax.experimental.pallas import tpu as pltpu
from jax.experimental.pallas import tpu_sc as plsc

B = 8
P = 1000
PP = 1024
H = 128
D = 64

NC = 2
NS = 16
NW = NC * NS

NBLK = B * PP // 128
BLK_PER_W = NBLK // NW
PRED_PER_W = BLK_PER_W * 128
HIST_PER_W = (B * H) // NW


def _sc_gather_body(w_item, pred_idx, hist_idx,
                    pred_out, hist_out,
                    pidx_v, prow_v, hidx_v, hrow_v, sem, semh):
    wid = lax.axis_index("s") * NC + lax.axis_index("c")
    lane16 = lax.broadcasted_iota(jnp.int32, (16,), 0)

    for j in range(BLK_PER_W):
        g = wid * BLK_PER_W + j
        b = lax.div(g, B)
        c = lax.rem(g, B)
        off = pl.multiple_of(c * 128, 128)
        pltpu.sync_copy(pred_idx.at[pl.ds(b, 1), pl.ds(off, 128)],
                        pidx_v.at[pl.ds(j, 1)])
    pltpu.sync_copy(hist_idx.at[pl.ds(wid, 1)], hidx_v)

    for j in range(BLK_PER_W):
        for gq in range(8):
            vec = pidx_v[j, gq * 16:(gq + 1) * 16]
            for t in range(16):
                sidx = jnp.sum(jnp.where(lane16 == t, vec, 0))
                pltpu.async_copy(
                    w_item.at[pl.ds(sidx, 1)],
                    prow_v.at[pl.ds(j * 128 + gq * 16 + t, 1)], sem)
    for gq in range(HIST_PER_W // 16):
        vec = hidx_v[0, gq * 16:(gq + 1) * 16]
        for t in range(16):
            sidx = jnp.sum(jnp.where(lane16 == t, vec, 0))
            pltpu.async_copy(w_item.at[pl.ds(sidx, 1)],
                             hrow_v.at[pl.ds(gq * 16 + t, 1)], semh)

    def pdrain(r, _):
        pltpu.make_async_copy(w_item.at[pl.ds(0, 1)],
                              prow_v.at[pl.ds(r, 1)], sem).wait()
        return 0
    lax.fori_loop(0, PRED_PER_W, pdrain, 0)

    def hdrain(r, _):
        pltpu.make_async_copy(w_item.at[pl.ds(0, 1)],
                              hrow_v.at[pl.ds(r, 1)], semh).wait()
        return 0
    lax.fori_loop(0, HIST_PER_W, hdrain, 0)

    pltpu.sync_copy(prow_v, pred_out.at[pl.ds(wid * PRED_PER_W, PRED_PER_W)])
    pltpu.sync_copy(hrow_v, hist_out.at[pl.ds(wid * HIST_PER_W, HIST_PER_W)])


def _make_sc_gather():
    return pl.kernel(
        _sc_gather_body,
        out_type=[
            jax.ShapeDtypeStruct((B * PP, D), jnp.float32),
            jax.ShapeDtypeStruct((B * H, D), jnp.float32),
        ],
        mesh=plsc.VectorSubcoreMesh(core_axis_name="c", subcore_axis_name="s",
                                    num_cores=NC, num_subcores=NS),
        compiler_params=pltpu.CompilerParams(needs_layout_passes=False),
        scratch_types=[
            pltpu.VMEM((BLK_PER_W, 128), jnp.int32),
            pltpu.VMEM((PRED_PER_W, D), jnp.float32),
            pltpu.VMEM((1, HIST_PER_W), jnp.int32),
            pltpu.VMEM((HIST_PER_W, D), jnp.float32),
            pltpu.SemaphoreType.DMA,
            pltpu.SemaphoreType.DMA,
        ],
    )


def _sc_scalars_body(ib_tbl, ul_tbl, ub_tbl, ib_idx, user_idx,
                     ib_out, ul_out, ub_out,
                     bidx_v, pbias_v, uidx_v, ulv, ubv, sem, semu):
    wid = lax.axis_index("s") * NC + lax.axis_index("c")
    pltpu.sync_copy(ib_idx.at[pl.ds(wid * BLK_PER_W, BLK_PER_W)], bidx_v)
    copies = []
    for j in range(BLK_PER_W):
        copies.append(pltpu.async_copy(ib_tbl.at[bidx_v.at[j]],
                                       pbias_v.at[j], sem))

    @pl.when(wid == 0)
    def _():
        pltpu.sync_copy(user_idx, uidx_v)
        cl = pltpu.async_copy(ul_tbl.at[uidx_v], ulv, semu)
        cb2 = pltpu.async_copy(ub_tbl.at[uidx_v], ubv, semu)
        cl.wait()
        cb2.wait()
        pltpu.sync_copy(ulv, ul_out)
        pltpu.sync_copy(ubv, ub_out)

    for c in copies:
        c.wait()
    pltpu.sync_copy(pbias_v, ib_out.at[pl.ds(wid * BLK_PER_W, BLK_PER_W)])


def _make_sc_scalars():
    return pl.kernel(
        _sc_scalars_body,
        out_type=[
            jax.ShapeDtypeStruct((NBLK, 128), jnp.float32),
            jax.ShapeDtypeStruct((B,), jnp.float32),
            jax.ShapeDtypeStruct((B,), jnp.float32),
        ],
        mesh=plsc.VectorSubcoreMesh(core_axis_name="c", subcore_axis_name="s",
                                    num_cores=NC, num_subcores=NS),
        compiler_params=pltpu.CompilerParams(use_tc_tiling_on_sc=False),
        scratch_types=[
            pltpu.VMEM((BLK_PER_W, 128), jnp.int32),
            pltpu.VMEM((BLK_PER_W, 128), jnp.float32),
            pltpu.VMEM((B,), jnp.int32),
            pltpu.VMEM((B,), jnp.float32),
            pltpu.VMEM((B,), jnp.float32),
            pltpu.SemaphoreType.DMA,
            pltpu.SemaphoreType.DMA,
        ],
    )


def _tc_dense_body(p2_ref, hist_ref, w_userT, uidx_ref, ib_ref, td_ref,
                   wt_ref, ul_ref, ub_ref, par_ref, out_ref,
                   ucol0, ucol1, ucol2, ucol3, ucol4, ucol5, ucol6, ucol7,
                   sem):
    ucols = [ucol0, ucol1, ucol2, ucol3, ucol4, ucol5, ucol6, ucol7]
    copies = []
    for i in range(B):
        base = pl.multiple_of((uidx_ref[i] // 128) * 128, 128)
        copies.append(pltpu.async_copy(
            w_userT.at[:, pl.ds(base, 128)], ucols[i], sem))
    for c in copies:
        c.wait()
    lanes = lax.broadcasted_iota(jnp.int32, (1, 128), 1)
    dn_t = (((1,), (1,)), ((), ()))
    user_all = jnp.concatenate(
        [lax.dot_general(
            (lanes == (uidx_ref[i] % 128)).astype(jnp.float32),
            ucols[i][:, :], dn_t,
            preferred_element_type=jnp.float32,
            precision=lax.Precision.HIGHEST)
         for i in range(B)], axis=0)

    glamb = par_ref[0]
    alpha = par_ref[1]
    beta = par_ref[2]
    gamma = par_ref[3]
    cutoff = par_ref[4]
    smooth = par_ref[5]
    force = par_ref[6]
    inv_denom = 1.0 + jnp.exp(force * smooth - smooth)
    ones_row = jnp.ones((1, D), jnp.float32)
    dn_s = (((1,), (0,)), ((), ()))
    for b in range(B):
        pred = p2_ref[b]
        hist = hist_ref[b]
        u = user_all[b:b + 1, :]
        pn = lax.dot_general(ones_row, pred * pred, dn_t,
                             preferred_element_type=jnp.float32,
                             precision=lax.Precision.HIGHEST)
        ph = lax.dot_general(hist, pred, dn_t,
                             preferred_element_type=jnp.float32,
                             precision=lax.Precision.HIGHEST)
        up = lax.dot_general(u, pred, dn_t,
                             preferred_element_type=jnp.float32,
                             precision=lax.Precision.HIGHEST)
        hn = jnp.sum(hist * hist, axis=1, keepdims=True)
        un = jnp.sum(u * u, axis=1, keepdims=True)
        dist = jnp.sqrt(jnp.maximum(hn + pn - 2.0 * ph, 0.0))
        sig = inv_denom / (1.0 + jnp.exp(force * smooth - smooth / (1.0 + dist)))
        coeff = ((glamb + ul_ref[b])
                 * lax.rsqrt(td_ref[b:b + 1, :] + cutoff)
                 * wt_ref[b:b + 1, :])
        res = lax.dot_general(coeff, sig, dn_s,
                              preferred_element_type=jnp.float32,
                              precision=lax.Precision.HIGHEST)
        du = jnp.sqrt(jnp.maximum(un + pn - 2.0 * up, 0.0))
        outp = jnp.maximum(du - res, 0.0)
        ib_row = jnp.concatenate(
            [ib_ref[8 * c + b:8 * c + b + 1, :] for c in range(PP // 128)],
            axis=1)
        out_ref[b:b + 1, :] = (alpha * outp + beta * outp * outp + gamma
                               + ub_ref[b] + ib_row)


def _tc_dense(p2, hist3, w_userT, uidx, ib_g, td, wt, ul_g, ub_g, params):
    return pl.pallas_call(
        _tc_dense_body,
        out_shape=jax.ShapeDtypeStruct((B, PP), jnp.float32),
        in_specs=[
            pl.BlockSpec(memory_space=pltpu.VMEM),
            pl.BlockSpec(memory_space=pltpu.VMEM),
            pl.BlockSpec(memory_space=pltpu.MemorySpace.HBM),
            pl.BlockSpec(memory_space=pltpu.SMEM),
            pl.BlockSpec(memory_space=pltpu.VMEM),
            pl.BlockSpec(memory_space=pltpu.VMEM),
            pl.BlockSpec(memory_space=pltpu.VMEM),
            pl.BlockSpec(memory_space=pltpu.SMEM),
            pl.BlockSpec(memory_space=pltpu.SMEM),
            pl.BlockSpec(memory_space=pltpu.SMEM),
        ],
        out_specs=pl.BlockSpec(memory_space=pltpu.VMEM),
        scratch_shapes=[pltpu.VMEM((D, 128), jnp.float32) for _ in range(B)]
        + [pltpu.SemaphoreType.DMA],
    )(p2, hist3, w_userT, uidx, ib_g, td, wt, ul_g, ub_g, params)


def kernel(history_timedeltas, history_weights, W_user, W_item, user_lamb,
           user_bias, item_bias, global_lamb, alpha, beta, gamma, cutoff,
           smooth, force, user_index, pred_item_indices,
           history_item_indices):
    pidx = jnp.pad(pred_item_indices.astype(jnp.int32), ((0, 0), (0, PP - P)))
    ib_idx = pidx.reshape(B, 8, 128).transpose(1, 0, 2).reshape(NBLK, 128)
    hidx = history_item_indices.astype(jnp.int32).reshape(NW, HIST_PER_W)
    uidx = user_index.astype(jnp.int32)
    pred_rows, hist_rows = _make_sc_gather()(W_item, pidx, hidx)
    ib_g, ul_g, ub_g = _make_sc_scalars()(
        item_bias.reshape(-1), user_lamb.reshape(-1), user_bias.reshape(-1),
        ib_idx, uidx)
    params = jnp.stack([global_lamb, alpha, beta, gamma, cutoff, smooth,
                        force]).astype(jnp.float32)
    out = _tc_dense(pred_rows.reshape(B, PP, D),
                    hist_rows.reshape(B, H, D), W_user.T, uidx, ib_g,
                    history_timedeltas, history_weights, ul_g, ub_g, params)
    return out[:, :P]


# speedup vs baseline: 2.0572x; 1.0014x over previous
import jax
import jax.numpy as jnp
from jax import lax
from jax.experimental import pallas as pl
from jax.experimental.pallas import tpu as pltpu
from jax.experimental.pallas import tpu_sc as plsc

B = 8
P = 1000
PP = 1024
H = 128
D = 64

NC = 2
NS = 16
NW = NC * NS

NBLK = B * PP // 128
BLK_PER_W = NBLK // NW
PRED_PER_W = BLK_PER_W * 128
HIST_PER_W = (B * H) // NW


def _sc_gather_body(w_item, pred_idx, hist_idx,
                    pred_out, hist_out,
                    pidx_v, prow_v, hidx_v, hrow_v, sem, semh):
    wid = lax.axis_index("s") * NC + lax.axis_index("c")
    lane16 = lax.broadcasted_iota(jnp.int32, (16,), 0)

    for j in range(BLK_PER_W):
        g = wid * BLK_PER_W + j
        b = lax.div(g, B)
        c = lax.rem(g, B)
        off = pl.multiple_of(c * 128, 128)
        pltpu.sync_copy(pred_idx.at[pl.ds(b, 1), pl.ds(off, 128)],
                        pidx_v.at[pl.ds(j, 1)])
    pltpu.sync_copy(hist_idx.at[pl.ds(wid, 1)], hidx_v)

    for j in range(BLK_PER_W):
        for gq in range(8):
            vec = pidx_v[j, gq * 16:(gq + 1) * 16]
            for t in range(16):
                sidx = jnp.sum(jnp.where(lane16 == t, vec, 0))
                pltpu.async_copy(
                    w_item.at[pl.ds(sidx, 1)],
                    prow_v.at[pl.ds(j * 128 + gq * 16 + t, 1)], sem)
    for gq in range(HIST_PER_W // 16):
        vec = hidx_v[0, gq * 16:(gq + 1) * 16]
        for t in range(16):
            sidx = jnp.sum(jnp.where(lane16 == t, vec, 0))
            pltpu.async_copy(w_item.at[pl.ds(sidx, 1)],
                             hrow_v.at[pl.ds(gq * 16 + t, 1)], semh)

    def pdrain(r, _):
        pltpu.make_async_copy(w_item.at[pl.ds(0, 1)],
                              prow_v.at[pl.ds(r, 1)], sem).wait()
        return 0
    lax.fori_loop(0, PRED_PER_W, pdrain, 0)

    def hdrain(r, _):
        pltpu.make_async_copy(w_item.at[pl.ds(0, 1)],
                              hrow_v.at[pl.ds(r, 1)], semh).wait()
        return 0
    lax.fori_loop(0, HIST_PER_W, hdrain, 0)

    pltpu.sync_copy(prow_v, pred_out.at[pl.ds(wid * PRED_PER_W, PRED_PER_W)])
    pltpu.sync_copy(hrow_v, hist_out.at[pl.ds(wid * HIST_PER_W, HIST_PER_W)])


def _make_sc_gather():
    return pl.kernel(
        _sc_gather_body,
        out_type=[
            jax.ShapeDtypeStruct((B * PP, D), jnp.float32),
            jax.ShapeDtypeStruct((B * H, D), jnp.float32),
        ],
        mesh=plsc.VectorSubcoreMesh(core_axis_name="c", subcore_axis_name="s",
                                    num_cores=NC, num_subcores=NS),
        compiler_params=pltpu.CompilerParams(needs_layout_passes=False),
        scratch_types=[
            pltpu.VMEM((BLK_PER_W, 128), jnp.int32),
            pltpu.VMEM((PRED_PER_W, D), jnp.float32),
            pltpu.VMEM((1, HIST_PER_W), jnp.int32),
            pltpu.VMEM((HIST_PER_W, D), jnp.float32),
            pltpu.SemaphoreType.DMA,
            pltpu.SemaphoreType.DMA,
        ],
    )


def _sc_scalars_body(ib_tbl, ul_tbl, ub_tbl, ib_idx, user_idx,
                     ib_out, ul_out, ub_out,
                     bidx_v, pbias_v, uidx_v, ulv, ubv, sem, semu):
    wid = lax.axis_index("s") * NC + lax.axis_index("c")
    pltpu.sync_copy(ib_idx.at[pl.ds(wid * BLK_PER_W, BLK_PER_W)], bidx_v)
    copies = []
    for j in range(BLK_PER_W):
        copies.append(pltpu.async_copy(ib_tbl.at[bidx_v.at[j]],
                                       pbias_v.at[j], sem))

    @pl.when(wid == 0)
    def _():
        pltpu.sync_copy(user_idx, uidx_v)
        cl = pltpu.async_copy(ul_tbl.at[uidx_v], ulv, semu)
        cb2 = pltpu.async_copy(ub_tbl.at[uidx_v], ubv, semu)
        cl.wait()
        cb2.wait()
        pltpu.sync_copy(ulv, ul_out)
        pltpu.sync_copy(ubv, ub_out)

    for c in copies:
        c.wait()
    pltpu.sync_copy(pbias_v, ib_out.at[pl.ds(wid * BLK_PER_W, BLK_PER_W)])


def _make_sc_scalars():
    return pl.kernel(
        _sc_scalars_body,
        out_type=[
            jax.ShapeDtypeStruct((NBLK, 128), jnp.float32),
            jax.ShapeDtypeStruct((B,), jnp.float32),
            jax.ShapeDtypeStruct((B,), jnp.float32),
        ],
        mesh=plsc.VectorSubcoreMesh(core_axis_name="c", subcore_axis_name="s",
                                    num_cores=NC, num_subcores=NS),
        compiler_params=pltpu.CompilerParams(use_tc_tiling_on_sc=False),
        scratch_types=[
            pltpu.VMEM((BLK_PER_W, 128), jnp.int32),
            pltpu.VMEM((BLK_PER_W, 128), jnp.float32),
            pltpu.VMEM((B,), jnp.int32),
            pltpu.VMEM((B,), jnp.float32),
            pltpu.VMEM((B,), jnp.float32),
            pltpu.SemaphoreType.DMA,
            pltpu.SemaphoreType.DMA,
        ],
    )


def _tc_dense_body(p2_ref, hist_ref, w_userT, uidx_ref, ib_ref, td_ref,
                   wt_ref, ul_ref, ub_ref, par_ref, out_ref,
                   ucol0, ucol1, ucol2, ucol3, ucol4, ucol5, ucol6, ucol7,
                   sem):
    ucols = [ucol0, ucol1, ucol2, ucol3, ucol4, ucol5, ucol6, ucol7]
    copies = []
    for i in range(B):
        base = pl.multiple_of((uidx_ref[i] // 128) * 128, 128)
        copies.append(pltpu.async_copy(
            w_userT.at[:, pl.ds(base, 128)], ucols[i], sem))
    for c in copies:
        c.wait()
    lanes = lax.broadcasted_iota(jnp.int32, (1, 128), 1)
    dn_t = (((1,), (1,)), ((), ()))
    user_all = jnp.concatenate(
        [lax.dot_general(
            (lanes == (uidx_ref[i] % 128)).astype(jnp.float32),
            ucols[i][:, :], dn_t,
            preferred_element_type=jnp.float32,
            precision=lax.Precision.HIGHEST)
         for i in range(B)], axis=0)

    glamb = par_ref[0]
    alpha = par_ref[1]
    beta = par_ref[2]
    gamma = par_ref[3]
    cutoff = par_ref[4]
    smooth = par_ref[5]
    force = par_ref[6]
    inv_denom = 1.0 + jnp.exp(force * smooth - smooth)
    ones_row = jnp.ones((1, D), jnp.float32)
    dn_s = (((1,), (0,)), ((), ()))
    for b in range(B):
        pred = p2_ref[b]
        hist = hist_ref[b]
        u = user_all[b:b + 1, :]
        pn = lax.dot_general(ones_row, pred * pred, dn_t,
                             preferred_element_type=jnp.float32,
                             precision=lax.Precision.HIGHEST)
        ph = lax.dot_general(hist, pred, dn_t,
                             preferred_element_type=jnp.float32,
                             precision=lax.Precision.HIGHEST)
        up = lax.dot_general(u, pred, dn_t,
                             preferred_element_type=jnp.float32,
                             precision=lax.Precision.HIGHEST)
        hn = jnp.sum(hist * hist, axis=1, keepdims=True)
        un = jnp.sum(u * u, axis=1, keepdims=True)
        dist = jnp.sqrt(jnp.maximum(hn + pn - 2.0 * ph, 0.0))
        sig = inv_denom / (1.0 + jnp.exp(force * smooth - smooth / (1.0 + dist)))
        coeff = ((glamb + ul_ref[b])
                 * lax.rsqrt(td_ref[b:b + 1, :] + cutoff)
                 * wt_ref[b:b + 1, :])
        res = lax.dot_general(coeff, sig, dn_s,
                              preferred_element_type=jnp.float32,
                              precision=lax.Precision.HIGHEST)
        du = jnp.sqrt(jnp.maximum(un + pn - 2.0 * up, 0.0))
        outp = jnp.maximum(du - res, 0.0)
        ib_row = jnp.concatenate(
            [ib_ref[8 * c + b:8 * c + b + 1, :] for c in range(PP // 128)],
            axis=1)
        out_ref[b:b + 1, :] = (alpha * outp + beta * outp * outp + gamma
                               + ub_ref[b] + ib_row)


def _tc_dense(p2, hist3, w_userT, uidx, ib_g, td, wt, ul_g, ub_g, params):
    return pl.pallas_call(
        _tc_dense_body,
        out_shape=jax.ShapeDtypeStruct((B, PP), jnp.float32),
        in_specs=[
            pl.BlockSpec(memory_space=pltpu.VMEM),
            pl.BlockSpec(memory_space=pltpu.VMEM),
            pl.BlockSpec(memory_space=pltpu.MemorySpace.HBM),
            pl.BlockSpec(memory_space=pltpu.SMEM),
            pl.BlockSpec(memory_space=pltpu.VMEM),
            pl.BlockSpec(memory_space=pltpu.VMEM),
            pl.BlockSpec(memory_space=pltpu.VMEM),
            pl.BlockSpec(memory_space=pltpu.SMEM),
            pl.BlockSpec(memory_space=pltpu.SMEM),
            pl.BlockSpec(memory_space=pltpu.SMEM),
        ],
        out_specs=pl.BlockSpec(memory_space=pltpu.VMEM),
        scratch_shapes=[pltpu.VMEM((D, 128), jnp.float32) for _ in range(B)]
        + [pltpu.SemaphoreType.DMA],
    )(p2, hist3, w_userT, uidx, ib_g, td, wt, ul_g, ub_g, params)


def kernel(history_timedeltas, history_weights, W_user, W_item, user_lamb,
           user_bias, item_bias, global_lamb, alpha, beta, gamma, cutoff,
           smooth, force, user_index, pred_item_indices,
           history_item_indices):
    pidx = jnp.pad(pred_item_indices.astype(jnp.int32), ((0, 0), (0, PP - P)))
    ib_idx = pidx.reshape(B, 8, 128).transpose(1, 0, 2).reshape(NBLK, 128)
    hidx = history_item_indices.astype(jnp.int32).reshape(NW, HIST_PER_W)
    uidx = user_index.astype(jnp.int32)
    ib_g, ul_g, ub_g = _make_sc_scalars()(
        item_bias.reshape(-1), user_lamb.reshape(-1), user_bias.reshape(-1),
        ib_idx, uidx)
    pred_rows, hist_rows = _make_sc_gather()(W_item, pidx, hidx)
    params = jnp.stack([global_lamb, alpha, beta, gamma, cutoff, smooth,
                        force]).astype(jnp.float32)
    out = _tc_dense(pred_rows.reshape(B, PP, D),
                    hist_rows.reshape(B, H, D), W_user.T, uidx, ib_g,
                    history_timedeltas, history_weights, ul_g, ub_g, params)
    return out[:, :P]


# speedup vs baseline: 2.2610x; 1.0990x over previous
import jax
import jax.numpy as jnp
from jax import lax
from jax.experimental import pallas as pl
from jax.experimental.pallas import tpu as pltpu
from jax.experimental.pallas import tpu_sc as plsc

B = 8
P = 1000
PP = 1024
H = 128
D = 64

NC = 2
NS = 16
NW = NC * NS

NBLK = B * PP // 128
BLK_PER_W = NBLK // NW
PRED_PER_W = BLK_PER_W * 128
HIST_PER_W = (B * H) // NW


def _sc_gather_body(w_item, pred_idx, hist_idx,
                    pred_out, hist_out,
                    pidx_v, prow_v, hidx_v, hrow_v, sem, semh):
    wid = lax.axis_index("s") * NC + lax.axis_index("c")
    lane16 = lax.broadcasted_iota(jnp.int32, (16,), 0)

    for j in range(BLK_PER_W):
        g = wid * BLK_PER_W + j
        b = lax.div(g, B)
        c = lax.rem(g, B)
        off = pl.multiple_of(c * 128, 128)
        pltpu.sync_copy(pred_idx.at[pl.ds(b, 1), pl.ds(off, 128)],
                        pidx_v.at[pl.ds(j, 1)])
    pltpu.sync_copy(hist_idx.at[pl.ds(wid, 1)], hidx_v)

    for j in range(BLK_PER_W):
        for gq in range(8):
            vec = pidx_v[j, gq * 16:(gq + 1) * 16]
            for t in range(16):
                sidx = jnp.sum(jnp.where(lane16 == t, vec, 0))
                pltpu.async_copy(
                    w_item.at[pl.ds(sidx, 1)],
                    prow_v.at[pl.ds(j * 128 + gq * 16 + t, 1)], sem)
    for gq in range(HIST_PER_W // 16):
        vec = hidx_v[0, gq * 16:(gq + 1) * 16]
        for t in range(16):
            sidx = jnp.sum(jnp.where(lane16 == t, vec, 0))
            pltpu.async_copy(w_item.at[pl.ds(sidx, 1)],
                             hrow_v.at[pl.ds(gq * 16 + t, 1)], semh)

    def pdrain(r, _):
        pltpu.make_async_copy(w_item.at[pl.ds(0, 1)],
                              prow_v.at[pl.ds(r, 1)], sem).wait()
        return 0
    lax.fori_loop(0, PRED_PER_W, pdrain, 0)

    def hdrain(r, _):
        pltpu.make_async_copy(w_item.at[pl.ds(0, 1)],
                              hrow_v.at[pl.ds(r, 1)], semh).wait()
        return 0
    lax.fori_loop(0, HIST_PER_W, hdrain, 0)

    pltpu.sync_copy(prow_v, pred_out.at[pl.ds(wid * PRED_PER_W, PRED_PER_W)])
    pltpu.sync_copy(hrow_v, hist_out.at[pl.ds(wid * HIST_PER_W, HIST_PER_W)])


def _make_sc_gather():
    return pl.kernel(
        _sc_gather_body,
        out_type=[
            jax.ShapeDtypeStruct((B * PP, D), jnp.float32),
            jax.ShapeDtypeStruct((B * H, D), jnp.float32),
        ],
        mesh=plsc.VectorSubcoreMesh(core_axis_name="c", subcore_axis_name="s",
                                    num_cores=NC, num_subcores=NS),
        compiler_params=pltpu.CompilerParams(needs_layout_passes=False),
        scratch_types=[
            pltpu.VMEM((BLK_PER_W, 128), jnp.int32),
            pltpu.VMEM((PRED_PER_W, D), jnp.float32),
            pltpu.VMEM((1, HIST_PER_W), jnp.int32),
            pltpu.VMEM((HIST_PER_W, D), jnp.float32),
            pltpu.SemaphoreType.DMA,
            pltpu.SemaphoreType.DMA,
        ],
    )


def _sc_scalars_body(ib_tbl, ul_tbl, ub_tbl, ib_idx, user_idx,
                     ib_out, ul_out, ub_out,
                     bidx_v, pbias_v, uidx_v, ulv, ubv, sem, semu):
    wid = lax.axis_index("s") * NC + lax.axis_index("c")
    pltpu.sync_copy(ib_idx.at[pl.ds(wid * BLK_PER_W, BLK_PER_W)], bidx_v)
    copies = []
    for j in range(BLK_PER_W):
        copies.append(pltpu.async_copy(ib_tbl.at[bidx_v.at[j]],
                                       pbias_v.at[j], sem))

    @pl.when(wid == 0)
    def _():
        pltpu.sync_copy(user_idx, uidx_v)
        cl = pltpu.async_copy(ul_tbl.at[uidx_v], ulv, semu)
        cb2 = pltpu.async_copy(ub_tbl.at[uidx_v], ubv, semu)
        cl.wait()
        cb2.wait()
        pltpu.sync_copy(ulv, ul_out)
        pltpu.sync_copy(ubv, ub_out)

    for c in copies:
        c.wait()
    pltpu.sync_copy(pbias_v, ib_out.at[pl.ds(wid * BLK_PER_W, BLK_PER_W)])


def _make_sc_scalars():
    return pl.kernel(
        _sc_scalars_body,
        out_type=[
            jax.ShapeDtypeStruct((NBLK, 128), jnp.float32),
            jax.ShapeDtypeStruct((B,), jnp.float32),
            jax.ShapeDtypeStruct((B,), jnp.float32),
        ],
        mesh=plsc.VectorSubcoreMesh(core_axis_name="c", subcore_axis_name="s",
                                    num_cores=NC, num_subcores=NS),
        compiler_params=pltpu.CompilerParams(use_tc_tiling_on_sc=False),
        scratch_types=[
            pltpu.VMEM((BLK_PER_W, 128), jnp.int32),
            pltpu.VMEM((BLK_PER_W, 128), jnp.float32),
            pltpu.VMEM((B,), jnp.int32),
            pltpu.VMEM((B,), jnp.float32),
            pltpu.VMEM((B,), jnp.float32),
            pltpu.SemaphoreType.DMA,
            pltpu.SemaphoreType.DMA,
        ],
    )


def _tc_dense_body(p2_ref, hist_ref, w_userT, uidx_ref, ib_ref, td_ref,
                   wt_ref, ul_ref, ub_ref, par_ref, out_ref,
                   ucol0, ucol1, ucol2, ucol3, ucol4, ucol5, ucol6, ucol7,
                   sem):
    ucols = [ucol0, ucol1, ucol2, ucol3, ucol4, ucol5, ucol6, ucol7]
    copies = []
    for i in range(B):
        base = pl.multiple_of((uidx_ref[i] // 128) * 128, 128)
        copies.append(pltpu.async_copy(
            w_userT.at[:, pl.ds(base, 128)], ucols[i], sem))
    for c in copies:
        c.wait()
    lanes = lax.broadcasted_iota(jnp.int32, (1, 128), 1)
    dn_t = (((1,), (1,)), ((), ()))
    user_all = jnp.concatenate(
        [lax.dot_general(
            (lanes == (uidx_ref[i] % 128)).astype(jnp.float32),
            ucols[i][:, :], dn_t,
            preferred_element_type=jnp.float32,
            precision=lax.Precision.DEFAULT)
         for i in range(B)], axis=0)

    glamb = par_ref[0]
    alpha = par_ref[1]
    beta = par_ref[2]
    gamma = par_ref[3]
    cutoff = par_ref[4]
    smooth = par_ref[5]
    force = par_ref[6]
    inv_denom = 1.0 + jnp.exp(force * smooth - smooth)
    ones_row = jnp.ones((1, D), jnp.float32)
    dn_s = (((1,), (0,)), ((), ()))
    for b in range(B):
        pred = p2_ref[b]
        hist = hist_ref[b]
        u = user_all[b:b + 1, :]
        pn = lax.dot_general(ones_row, pred * pred, dn_t,
                             preferred_element_type=jnp.float32,
                             precision=lax.Precision.DEFAULT)
        ph = lax.dot_general(hist, pred, dn_t,
                             preferred_element_type=jnp.float32,
                             precision=lax.Precision.DEFAULT)
        up = lax.dot_general(u, pred, dn_t,
                             preferred_element_type=jnp.float32,
                             precision=lax.Precision.DEFAULT)
        hn = jnp.sum(hist * hist, axis=1, keepdims=True)
        un = jnp.sum(u * u, axis=1, keepdims=True)
        dist = jnp.sqrt(jnp.maximum(hn + pn - 2.0 * ph, 0.0))
        sig = inv_denom / (1.0 + jnp.exp(force * smooth - smooth / (1.0 + dist)))
        coeff = ((glamb + ul_ref[b])
                 * lax.rsqrt(td_ref[b:b + 1, :] + cutoff)
                 * wt_ref[b:b + 1, :])
        res = lax.dot_general(coeff, sig, dn_s,
                              preferred_element_type=jnp.float32,
                              precision=lax.Precision.DEFAULT)
        du = jnp.sqrt(jnp.maximum(un + pn - 2.0 * up, 0.0))
        outp = jnp.maximum(du - res, 0.0)
        ib_row = jnp.concatenate(
            [ib_ref[8 * c + b:8 * c + b + 1, :] for c in range(PP // 128)],
            axis=1)
        out_ref[b:b + 1, :] = (alpha * outp + beta * outp * outp + gamma
                               + ub_ref[b] + ib_row)


def _tc_dense(p2, hist3, w_userT, uidx, ib_g, td, wt, ul_g, ub_g, params):
    return pl.pallas_call(
        _tc_dense_body,
        out_shape=jax.ShapeDtypeStruct((B, PP), jnp.float32),
        in_specs=[
            pl.BlockSpec(memory_space=pltpu.VMEM),
            pl.BlockSpec(memory_space=pltpu.VMEM),
            pl.BlockSpec(memory_space=pltpu.MemorySpace.HBM),
            pl.BlockSpec(memory_space=pltpu.SMEM),
            pl.BlockSpec(memory_space=pltpu.VMEM),
            pl.BlockSpec(memory_space=pltpu.VMEM),
            pl.BlockSpec(memory_space=pltpu.VMEM),
            pl.BlockSpec(memory_space=pltpu.SMEM),
            pl.BlockSpec(memory_space=pltpu.SMEM),
            pl.BlockSpec(memory_space=pltpu.SMEM),
        ],
        out_specs=pl.BlockSpec(memory_space=pltpu.VMEM),
        scratch_shapes=[pltpu.VMEM((D, 128), jnp.float32) for _ in range(B)]
        + [pltpu.SemaphoreType.DMA],
    )(p2, hist3, w_userT, uidx, ib_g, td, wt, ul_g, ub_g, params)


def kernel(history_timedeltas, history_weights, W_user, W_item, user_lamb,
           user_bias, item_bias, global_lamb, alpha, beta, gamma, cutoff,
           smooth, force, user_index, pred_item_indices,
           history_item_indices):
    pidx = jnp.pad(pred_item_indices.astype(jnp.int32), ((0, 0), (0, PP - P)))
    ib_idx = pidx.reshape(B, 8, 128).transpose(1, 0, 2).reshape(NBLK, 128)
    hidx = history_item_indices.astype(jnp.int32).reshape(NW, HIST_PER_W)
    uidx = user_index.astype(jnp.int32)
    ib_g, ul_g, ub_g = _make_sc_scalars()(
        item_bias.reshape(-1), user_lamb.reshape(-1), user_bias.reshape(-1),
        ib_idx, uidx)
    pred_rows, hist_rows = _make_sc_gather()(W_item, pidx, hidx)
    params = jnp.stack([global_lamb, alpha, beta, gamma, cutoff, smooth,
                        force]).astype(jnp.float32)
    out = _tc_dense(pred_rows.reshape(B, PP, D),
                    hist_rows.reshape(B, H, D), W_user.T, uidx, ib_g,
                    history_timedeltas, history_weights, ul_g, ub_g, params)
    return out[:, :P]
